# Initial kernel scaffold; baseline (speedup 1.0000x reference)
import jax
import jax.numpy as jnp
from jax.experimental import pallas as pl


def kernel(all_image_features, all_text_features, logits_per_image, logits_per_text, proj_w, proj_b):
    raise NotImplementedError("write your pallas kernel here")


# speedup vs baseline: 1.2326x; 1.2326x over previous
import functools

import jax
import jax.numpy as jnp
from jax.experimental import pallas as pl
from jax.experimental.pallas import tpu as pltpu

B = 4096
D = 512
R = 256
NBLK = B // R
PAD = 128


def _sample_project_body(li_ref, g1_ref, lt_ref, g2_ref, ai_ref, at_ref,
                         pwi_ref, pwt_ref,
                         idxt_ref, idxi_ref, pi_ref, pt_ref):
    i = pl.program_id(0)
    r0 = i * R

    col = jax.lax.broadcasted_iota(jnp.int32, (R, B), 1)
    row = r0 + jax.lax.broadcasted_iota(jnp.int32, (R, B), 0)
    diag = col == row

    def draw(x, g):
        m = jnp.max(x, axis=1, keepdims=True)
        u = jnp.exp(x - m)
        s = jnp.sum(u, axis=1, keepdims=True)
        w = u / s
        w = jnp.where(diag, 0.0, w)
        v = jnp.where(w > 0, jnp.log(w), -jnp.inf) + g
        vmax = jnp.max(v, axis=1, keepdims=True)
        cand = jnp.where(v == vmax, col, B)
        return jnp.min(cand, axis=1).astype(jnp.int32)

    idxt_ref[0, pl.ds(r0, R)] = draw(li_ref[...], g1_ref[...])
    idxi_ref[0, pl.ds(r0, R)] = draw(lt_ref[...], g2_ref[...])

    pi_ref[...] = jnp.dot(ai_ref[...], pwi_ref[...],
                          preferred_element_type=jnp.float32)
    pt_ref[...] = jnp.dot(at_ref[...], pwt_ref[...],
                          preferred_element_type=jnp.float32)


def _finalize_body(idxt_ref, idxi_ref, pi_ref, pt_ref, pb_ref,
                   lg0_ref, lg1_ref, lg2_ref, loss_ref):
    i = pl.program_id(0)
    r0 = i * R

    idx_t = idxt_ref[0, pl.ds(r0, R)]
    idx_i = idxi_ref[0, pl.ds(r0, R)]
    col = jax.lax.broadcasted_iota(jnp.int32, (R, B), 1)
    oh_t = (col == idx_t[:, None]).astype(jnp.float32)
    oh_i = (col == idx_i[:, None]).astype(jnp.float32)
    gath_t = jnp.dot(oh_t, pt_ref[...], preferred_element_type=jnp.float32)
    gath_i = jnp.dot(oh_i, pi_ref[...], preferred_element_type=jnp.float32)

    pi_blk = pi_ref[pl.ds(r0, R), :]
    pt_blk = pt_ref[pl.ds(r0, R), :]
    pb = pb_ref[...]

    lg0 = pi_blk + pt_blk + pb
    lg1 = pi_blk + gath_t + pb
    lg2 = gath_i + pt_blk + pb
    lg0_ref[...] = lg0
    lg1_ref[...] = lg1
    lg2_ref[...] = lg2

    def logp(lg, want_pos):
        a = lg[:, 0:1]
        b = lg[:, 1:2]
        mx = jnp.maximum(a, b)
        lse = jnp.log(jnp.exp(a - mx) + jnp.exp(b - mx))
        sel = b if want_pos else a
        return (sel - mx) - lse

    partial = (jnp.sum(logp(lg0, True)) + jnp.sum(logp(lg1, False))
               + jnp.sum(logp(lg2, False)))

    @pl.when(i == 0)
    def _():
        loss_ref[...] = jnp.zeros_like(loss_ref)

    loss_ref[...] += jnp.full((1, 1), partial, jnp.float32)

    @pl.when(i == NBLK - 1)
    def _():
        loss_ref[...] = loss_ref[...] * (-1.0 / (3.0 * B))


@functools.partial(jax.jit, static_argnames=())
def kernel(all_image_features, all_text_features, logits_per_image,
           logits_per_text, proj_w, proj_b):
    skey = jax.random.key(42)
    k1, k2 = jax.random.split(skey)
    g1 = jax.random.gumbel(k1, (B, B), jnp.float32)
    g2 = jax.random.gumbel(k2, (B, B), jnp.float32)

    pw_img = jnp.zeros((D, PAD), jnp.float32).at[:, :2].set(proj_w[:D])
    pw_txt = jnp.zeros((D, PAD), jnp.float32).at[:, :2].set(proj_w[D:])
    pb_pad = jnp.zeros((1, PAD), jnp.float32).at[0, :2].set(proj_b)

    row_spec = pl.BlockSpec((R, B), lambda i: (i, 0))
    feat_spec = pl.BlockSpec((R, D), lambda i: (i, 0))
    full_w = pl.BlockSpec((D, PAD), lambda i: (0, 0))
    idx_spec = pl.BlockSpec((1, B), lambda i: (0, 0))
    proj_out = pl.BlockSpec((R, PAD), lambda i: (i, 0))

    idxt, idxi, pi, pt = pl.pallas_call(
        _sample_project_body,
        grid=(NBLK,),
        in_specs=[row_spec, row_spec, row_spec, row_spec,
                  feat_spec, feat_spec, full_w, full_w],
        out_specs=[idx_spec, idx_spec, proj_out, proj_out],
        out_shape=[
            jax.ShapeDtypeStruct((1, B), jnp.int32),
            jax.ShapeDtypeStruct((1, B), jnp.int32),
            jax.ShapeDtypeStruct((B, PAD), jnp.float32),
            jax.ShapeDtypeStruct((B, PAD), jnp.float32),
        ],
    )(logits_per_image, g1, logits_per_text, g2,
      all_image_features, all_text_features, pw_img, pw_txt)

    full_proj = pl.BlockSpec((B, PAD), lambda i: (0, 0))
    pb_spec = pl.BlockSpec((1, PAD), lambda i: (0, 0))
    lg_spec = pl.BlockSpec((R, PAD), lambda i: (i, 0))
    loss_spec = pl.BlockSpec((1, 1), lambda i: (0, 0))

    lg0, lg1, lg2, loss = pl.pallas_call(
        _finalize_body,
        grid=(NBLK,),
        in_specs=[idx_spec, idx_spec, full_proj, full_proj, pb_spec],
        out_specs=[lg_spec, lg_spec, lg_spec, loss_spec],
        out_shape=[
            jax.ShapeDtypeStruct((B, PAD), jnp.float32),
            jax.ShapeDtypeStruct((B, PAD), jnp.float32),
            jax.ShapeDtypeStruct((B, PAD), jnp.float32),
            jax.ShapeDtypeStruct((1, 1), jnp.float32),
        ],
    )(idxt, idxi, pi, pt, pb_pad)

    logits = jnp.concatenate([lg0[:, :2], lg1[:, :2], lg2[:, :2]], axis=0)
    itm_labels = jnp.concatenate([
        jnp.ones((B,), dtype=jnp.int32),
        jnp.zeros((B,), dtype=jnp.int32),
        jnp.zeros((B,), dtype=jnp.int32),
    ])
    return loss[0, 0], logits, itm_labels
